# baseline (speedup 1.0000x reference)
import functools

import numpy as np

import jax
import jax.numpy as jnp
from jax import lax
from jax.experimental import pallas as pl
from jax.experimental.pallas import tpu as pltpu
from jax.experimental.pallas import tpu_sc as plsc

N = 10000
E = 320000
D_IN = 128
D_OUT = 128
H = 2 * D_OUT

B = 512
NPAD = 10240
NBLK = NPAD // B

NSUB = 16
EPAD = 322560
C = 96
CHUNKS1 = EPAD // (NSUB * C)
CHUNKS2 = EPAD // (2 * NSUB * C)
H_ACC = 10112
ROWS_PER_SUB = H_ACC // NSUB
E_PER_TILE = EPAD // 32


KA = D_IN + 8


def _pack_bf16_pairs(lo, hi):
    bl = lax.bitcast_convert_type(lo, jnp.int32)
    bh = lax.bitcast_convert_type(hi, jnp.int32)
    rl = lax.shift_right_logical(
        bl + 0x7FFF + lax.bitwise_and(lax.shift_right_logical(bl, 16), 1), 16)
    rh = lax.shift_right_logical(
        bh + 0x7FFF + lax.bitwise_and(lax.shift_right_logical(bh, 16), 1), 16)
    return lax.bitwise_or(lax.bitwise_and(rl, 0xFFFF),
                          lax.shift_left(rh, 16))


def _tc_pre1_body(x_ref, wlin_ref, wfb_ref, wfg_ref,
                  wskip_ref, wfsb_ref, wfsg_ref,
                  xl_ref, gb_ref, skip_ref):
    x = x_ref[...]
    dot = functools.partial(jnp.dot, preferred_element_type=jnp.float32)
    xl_ref[0] = dot(x, wlin_ref[...])
    beta = dot(x, wfb_ref[...])
    gamma = dot(x, wfg_ref[...])
    gb_ref[0] = jnp.concatenate(
        (_pack_bf16_pairs(gamma[:, :64], gamma[:, 64:]),
         _pack_bf16_pairs(beta[:, :64], beta[:, 64:])), axis=1)
    beta_s = dot(x, wfsb_ref[...])
    gamma_s = dot(x, wfsg_ref[...])
    skip_ref[...] = jnp.maximum(gamma_s * dot(x, wskip_ref[...]) + beta_s, 0.0)


def _tc_pre1(x_aug, W_lin1a, W_film1a, W_skip1a, W_fskip1a):
    half = D_OUT
    return pl.pallas_call(
        _tc_pre1_body,
        grid=(NBLK, 2),
        in_specs=[
            pl.BlockSpec((B, KA), lambda n, c: (n, 0)),
            pl.BlockSpec((KA, half), lambda n, c: (0, c)),
            pl.BlockSpec((KA, half), lambda n, c: (0, c)),
            pl.BlockSpec((KA, half), lambda n, c: (0, 2 + c)),
            pl.BlockSpec((KA, half), lambda n, c: (0, c)),
            pl.BlockSpec((KA, half), lambda n, c: (0, c)),
            pl.BlockSpec((KA, half), lambda n, c: (0, 2 + c)),
        ],
        out_specs=[
            pl.BlockSpec((1, B, half), lambda n, c: (c, n, 0)),
            pl.BlockSpec((1, B, half), lambda n, c: (c, n, 0)),
            pl.BlockSpec((B, half), lambda n, c: (n, c)),
        ],
        out_shape=[
            jax.ShapeDtypeStruct((2, NPAD, half), jnp.float32),
            jax.ShapeDtypeStruct((2, NPAD, half), jnp.int32),
            jax.ShapeDtypeStruct((NPAD, H), jnp.float32),
        ],
    )(x_aug, W_lin1a, W_film1a, W_film1a, W_skip1a, W_fskip1a, W_fskip1a)


def _tc_mid_body(acc_ref, cnt_ref, skip_ref, wlin_ref, wfilm_ref, bf_ref,
                 wskip_ref, wfskip_ref,
                 xl2_ref, gb2_ref, skip2_ref):
    dot = functools.partial(jnp.dot, preferred_element_type=jnp.float32)
    summed = jnp.concatenate((acc_ref[0], acc_ref[1]), axis=1)
    cnt = jnp.sum(cnt_ref[:, 0], axis=(0, 1)).reshape(B, 1)
    h = jnp.maximum(skip_ref[...] + summed / jnp.clip(cnt, 1.0, None), 0.0)
    xl2_ref[...] = dot(h, wlin_ref[...])
    f2 = dot(h, wfilm_ref[...]) + bf_ref[...]
    gb2_ref[...] = jnp.concatenate(
        (_pack_bf16_pairs(f2[:, 128:192], f2[:, 192:]),
         _pack_bf16_pairs(f2[:, :64], f2[:, 64:128])), axis=1)
    fs2 = dot(h, wfskip_ref[...])
    skip2_ref[...] = jnp.maximum(
        fs2[:, 128:] * dot(h, wskip_ref[...]) + fs2[:, :128], 0.0)


def _tc_mid(acc1, cnt, skip1, W_lin2, W_film2, b_film2, W_skip2, W_fskip2):
    b2 = b_film2.reshape(1, 2 * D_OUT)
    return pl.pallas_call(
        _tc_mid_body,
        grid=(NBLK,),
        in_specs=[
            pl.BlockSpec((2, B, 128), lambda n: (0, n, 0)),
            pl.BlockSpec((2, 1, NSUB, B), lambda n: (0, n, 0, 0)),
            pl.BlockSpec((B, H), lambda n: (n, 0)),
            pl.BlockSpec((H, D_OUT), lambda n: (0, 0)),
            pl.BlockSpec((H, 2 * D_OUT), lambda n: (0, 0)),
            pl.BlockSpec((1, 2 * D_OUT), lambda n: (0, 0)),
            pl.BlockSpec((H, D_OUT), lambda n: (0, 0)),
            pl.BlockSpec((H, 2 * D_OUT), lambda n: (0, 0)),
        ],
        out_specs=[
            pl.BlockSpec((B, D_OUT), lambda n: (n, 0)),
            pl.BlockSpec((B, D_OUT), lambda n: (n, 0)),
            pl.BlockSpec((B, D_OUT), lambda n: (n, 0)),
        ],
        out_shape=[
            jax.ShapeDtypeStruct((NPAD, D_OUT), jnp.float32),
            jax.ShapeDtypeStruct((NPAD, D_OUT), jnp.int32),
            jax.ShapeDtypeStruct((NPAD, D_OUT), jnp.float32),
        ],
    )(acc1, cnt, skip1, W_lin2, W_film2, b2, W_skip2, W_fskip2)


def _tc_final_body(acc2_ref, skip2_ref, cnt_ref, out_ref):
    summed = acc2_ref[0] + acc2_ref[1]
    cnt = jnp.sum(cnt_ref[:, 0], axis=(0, 1)).reshape(B, 1)
    out_ref[...] = skip2_ref[...] + summed / jnp.clip(cnt, 1.0, None)


def _tc_final(acc2, skip2, cnt):
    return pl.pallas_call(
        _tc_final_body,
        grid=(NBLK,),
        in_specs=[
            pl.BlockSpec((2, B, D_OUT), lambda n: (0, n, 0)),
            pl.BlockSpec((B, D_OUT), lambda n: (n, 0)),
            pl.BlockSpec((2, 1, NSUB, B), lambda n: (0, n, 0, 0)),
        ],
        out_specs=pl.BlockSpec((B, D_OUT), lambda n: (n, 0)),
        out_shape=jax.ShapeDtypeStruct((NPAD, D_OUT), jnp.float32),
    )(acc2, skip2, cnt)


def _make_sc_agg(feature_split):
    mesh = plsc.VectorSubcoreMesh(
        core_axis_name="c", subcore_axis_name="s", num_cores=2,
        num_subcores=NSUB)
    chunks = CHUNKS1 if feature_split else CHUNKS2

    def body(epk_hbm, xl_hbm, gb_hbm, out_hbm,
             eidx0, eidx1, didxs0, didxs1, xl0, gb0, xl1, gb1,
             acc_sh, semx0, semg0, semx1, semg1, sems0, sems1):
        c = lax.axis_index("c")
        s = lax.axis_index("s")
        zeros16 = jnp.zeros((16,), jnp.float32)

        def zrow(i, carry):
            for j in range(128 // 16):
                xl0[i, pl.ds(j * 16, 16)] = zeros16
            return carry
        lax.fori_loop(0, C, zrow, 0)
        full = ROWS_PER_SUB // C
        for t in range(full):
            pltpu.sync_copy(xl0, acc_sh.at[pl.ds(s * ROWS_PER_SUB + t * C, C)])
        rem = ROWS_PER_SUB - full * C
        if rem:
            pltpu.sync_copy(
                xl0.at[pl.ds(0, rem)],
                acc_sh.at[pl.ds(s * ROWS_PER_SUB + full * C, rem)])
        plsc.subcore_barrier()

        if feature_split:
            base = s * (chunks * C)
            estride = C
            coff = c * NPAD
        else:
            base = s * (2 * chunks * C) + c * C
            estride = 2 * C
            coff = None

        def drain_scatter(xl_v, sems):
            pltpu.make_async_copy(xl_v, acc_sh.at[pl.ds(0, C)], sems).wait()

        def fetch(g, eidx_v, xl_v, gb_v, semx, semg,
                  sems=None, scat_pred=None):
            e0 = base + g * estride
            pltpu.sync_copy(epk_hbm.at[pl.ds(2 * e0, 2 * C)], eidx_v)
            if feature_split:
                def adj(i, cr):
                    eidx_v[pl.ds(i * 16, 16)] = eidx_v[pl.ds(i * 16, 16)] + coff
                    return cr
                lax.fori_loop(0, 2 * C // 16, adj, 0)
            if sems is not None:
                if scat_pred is None:
                    drain_scatter(xl_v, sems)
                else:
                    @pl.when(scat_pred)
                    def _():
                        drain_scatter(xl_v, sems)
            pltpu.async_copy(xl_hbm.at[eidx_v.at[pl.ds(0, C)]], xl_v, semx)
            pltpu.async_copy(gb_hbm.at[eidx_v.at[pl.ds(C, C)]], gb_v, semg)

        def drain(xl_v, gb_v, semx, semg):
            pltpu.make_async_copy(xl_hbm.at[pl.ds(0, C)], xl_v, semx).wait()
            pltpu.make_async_copy(gb_hbm.at[pl.ds(0, C)], gb_v, semg).wait()

        himask = jnp.int32(-65536)

        def compute(didx_v, xl_v, gb_v):
            def mrow(i, cr):
                for j in range(4):
                    wg = gb_v[i, pl.ds(j * 16, 16)]
                    wb = gb_v[i, pl.ds(64 + j * 16, 16)]
                    glo = plsc.bitcast(lax.shift_left(wg, 16), jnp.float32)
                    blo = plsc.bitcast(lax.shift_left(wb, 16), jnp.float32)
                    ghi = plsc.bitcast(lax.bitwise_and(wg, himask),
                                       jnp.float32)
                    bhi = plsc.bitcast(lax.bitwise_and(wb, himask),
                                       jnp.float32)
                    xlo = xl_v[i, pl.ds(32 * j, 16)]
                    xhi = xl_v[i, pl.ds(32 * j + 16, 16)]
                    xl_v[i, pl.ds(32 * j, 16)] = jnp.maximum(
                        glo * xlo + blo, 0.0)
                    xl_v[i, pl.ds(32 * j + 16, 16)] = jnp.maximum(
                        ghi * xhi + bhi, 0.0)
                return cr
            lax.fori_loop(0, C, mrow, 0)

        def scatter(eidx_v, didxs_v, xl_v, sems):
            def dcp(i, cr):
                d16 = eidx_v[pl.ds(C + i * 16, 16)]
                didxs_v[pl.ds(i * 16, 16)] = (d16 - coff) if feature_split else d16
                return cr
            lax.fori_loop(0, C // 16, dcp, 0)
            pltpu.make_async_copy(
                xl_v, acc_sh.at[didxs_v], sems).start(add=True)

        fetch(0, eidx0, xl0, gb0, semx0, semg0)

        def pair(g2, carry):
            g = 2 * g2
            fetch(g + 1, eidx1, xl1, gb1, semx1, semg1,
                  sems1, scat_pred=g2 > 0)
            drain(xl0, gb0, semx0, semg0)
            compute(eidx0, xl0, gb0)
            scatter(eidx0, didxs0, xl0, sems0)

            @pl.when(g + 2 < chunks)
            def _():
                fetch(g + 2, eidx0, xl0, gb0, semx0, semg0, sems0)
            drain(xl1, gb1, semx1, semg1)
            compute(eidx1, xl1, gb1)
            scatter(eidx1, didxs1, xl1, sems1)
            return carry
        lax.fori_loop(0, chunks // 2, pair, 0)
        if chunks % 2:
            drain(xl0, gb0, semx0, semg0)
            compute(eidx0, xl0, gb0)
            scatter(eidx0, didxs0, xl0, sems0)
        drain_scatter(xl0, sems0)
        drain_scatter(xl1, sems1)
        plsc.subcore_barrier()
        pltpu.sync_copy(acc_sh.at[pl.ds(s * ROWS_PER_SUB, ROWS_PER_SUB)],
                        out_hbm.at[c, pl.ds(s * ROWS_PER_SUB, ROWS_PER_SUB)])

    return pl.kernel(
        body,
        out_type=jax.ShapeDtypeStruct((2, NPAD, 128), jnp.float32),
        mesh=mesh,
        compiler_params=pltpu.CompilerParams(needs_layout_passes=False),
        scratch_types=(
            [pltpu.VMEM((2 * C,), jnp.int32)] * 2
            + [pltpu.VMEM((C,), jnp.int32)] * 2
            + [pltpu.VMEM((C, 128), jnp.float32),
               pltpu.VMEM((C, 128), jnp.int32)] * 2
            + [pltpu.VMEM_SHARED((H_ACC, 128), jnp.float32)]
            + [pltpu.SemaphoreType.DMA] * 6
        ),
    )


def _make_sc_count():
    mesh = plsc.VectorSubcoreMesh(
        core_axis_name="c", subcore_axis_name="s", num_cores=2,
        num_subcores=NSUB)

    def body(dst_hbm, cnt_hbm, didx_v, hist_v):
        c = lax.axis_index("c")
        s = lax.axis_index("s")
        w = c * NSUB + s
        zeros16 = jnp.zeros((16,), jnp.float32)
        iota16 = lax.iota(jnp.int32, 16)
        part16 = lax.bitwise_and(iota16, 3) * NPAD

        def hz(i, cr):
            hist_v[pl.ds(i * 16, 16)] = zeros16
            return cr
        lax.fori_loop(0, 4 * NPAD // 16, hz, 0)
        pltpu.sync_copy(dst_hbm.at[pl.ds(w * E_PER_TILE, E_PER_TILE)], didx_v)

        def step(i, cr):
            d16 = didx_v[pl.ds(i * 16, 16)]
            addr = part16 + d16
            for p in range(4):
                msk = jnp.logical_and(iota16 >= 4 * p, iota16 < 4 * p + 4)
                old = plsc.load_gather(hist_v, [addr], mask=msk)
                plsc.store_scatter(hist_v, [addr], old + 1.0, mask=msk)
            return cr
        lax.fori_loop(0, E_PER_TILE // 16, step, 0)

        def hred(i, cr):
            v = (hist_v[pl.ds(i * 16, 16)]
                 + hist_v[pl.ds(NPAD + i * 16, 16)]
                 + hist_v[pl.ds(2 * NPAD + i * 16, 16)]
                 + hist_v[pl.ds(3 * NPAD + i * 16, 16)])
            hist_v[pl.ds(i * 16, 16)] = v
            return cr
        lax.fori_loop(0, NPAD // 16, hred, 0)
        for n in range(NBLK):
            pltpu.sync_copy(hist_v.at[pl.ds(n * B, B)], cnt_hbm.at[c, n, s])

    return pl.kernel(
        body,
        out_type=jax.ShapeDtypeStruct((2, NBLK, NSUB, B), jnp.float32),
        mesh=mesh,
        compiler_params=pltpu.CompilerParams(needs_layout_passes=False),
        scratch_types=[
            pltpu.VMEM((E_PER_TILE,), jnp.int32),
            pltpu.VMEM((4 * NPAD,), jnp.float32),
        ],
    )


_sc_agg_cached = functools.lru_cache(maxsize=None)(_make_sc_agg)
_sc_count_cached = functools.lru_cache(maxsize=None)(_make_sc_count)


@jax.jit
def kernel(x, edge_index, W_lin1, W_film1, b_film1, W_skip1, W_fskip1,
           W_lin2, W_film2, b_film2, W_skip2, W_fskip2):
    x = x.astype(jnp.float32)
    src_pad = jnp.concatenate(
        [edge_index[0].astype(jnp.int32), jnp.full((EPAD - E,), N, jnp.int32)])
    dst_pad = jnp.concatenate(
        [edge_index[1].astype(jnp.int32), jnp.full((EPAD - E,), N, jnp.int32)])
    x_pad = jnp.pad(x, ((0, NPAD - N), (0, 0)))
    ones_col = jnp.ones((NPAD, 1), jnp.float32)
    x_aug = jnp.concatenate(
        [x_pad, ones_col, jnp.zeros((NPAD, KA - D_IN - 1), jnp.float32)], axis=1)

    def _aug_w(w, bias=None):
        pad = jnp.zeros((KA - D_IN, w.shape[1]), jnp.float32)
        if bias is not None:
            pad = pad.at[0].set(bias)
        return jnp.concatenate([w, pad], axis=0)

    def _perm(n):
        q = np.arange(n)
        b, r = q - q % 128, q % 128
        h, r2 = r // 64, r % 64
        return b + 32 * (r2 // 16) + 16 * h + r2 % 16

    p512 = _perm(512)
    xl1, gb1, skip1 = _tc_pre1(x_aug, _aug_w(W_lin1),
                               _aug_w(W_film1[:, p512], b_film1[p512]),
                               _aug_w(W_skip1), _aug_w(W_fskip1))
    cnt1 = _sc_count_cached()(dst_pad)
    epk = jnp.stack([src_pad.reshape(-1, C), dst_pad.reshape(-1, C)],
                    axis=1).reshape(-1)
    acc1 = _sc_agg_cached(True)(
        epk, xl1.reshape(2 * NPAD, 128), gb1.reshape(2 * NPAD, 128))
    pblk = np.concatenate([_perm(128), 128 + _perm(128)])
    xl2, gb2, skip2 = _tc_mid(acc1, cnt1, skip1, W_lin2,
                              W_film2[:, pblk], b_film2[pblk],
                              W_skip2, W_fskip2)
    acc2 = _sc_agg_cached(False)(epk, xl2, gb2)
    out = _tc_final(acc2, skip2, cnt1)
    return out[:N]


# speedup vs baseline: 1.1931x; 1.1931x over previous
import functools

import numpy as np

import jax
import jax.numpy as jnp
from jax import lax
from jax.experimental import pallas as pl
from jax.experimental.pallas import tpu as pltpu
from jax.experimental.pallas import tpu_sc as plsc

N = 10000
E = 320000
D_IN = 128
D_OUT = 128
H = 2 * D_OUT

B = 512
NPAD = 10240
NBLK = NPAD // B

NSUB = 16
EPAD = E
C = 80
CHUNKS1 = EPAD // (NSUB * C)
CHUNKS2 = EPAD // (2 * NSUB * C)
H_ACC = 10112
ROWS_PER_SUB = H_ACC // NSUB
E_PER_TILE = EPAD // 32


KA = D_IN + 8


def _pack_bf16_pairs(lo, hi):
    bl = lax.bitcast_convert_type(lo, jnp.int32)
    bh = lax.bitcast_convert_type(hi, jnp.int32)
    rl = lax.shift_right_logical(
        bl + 0x7FFF + lax.bitwise_and(lax.shift_right_logical(bl, 16), 1), 16)
    rh = lax.shift_right_logical(
        bh + 0x7FFF + lax.bitwise_and(lax.shift_right_logical(bh, 16), 1), 16)
    return lax.bitwise_or(lax.bitwise_and(rl, 0xFFFF),
                          lax.shift_left(rh, 16))


def _tc_pre1_body(x_ref, wlin_ref, wfb_ref, wfg_ref,
                  wskip_ref, wfsb_ref, wfsg_ref,
                  xl_ref, gb_ref, skip_ref):
    x = x_ref[...]
    dot = functools.partial(jnp.dot, preferred_element_type=jnp.float32)
    xl_ref[0] = dot(x, wlin_ref[...])
    beta = dot(x, wfb_ref[...])
    gamma = dot(x, wfg_ref[...])
    gb_ref[0] = jnp.concatenate(
        (_pack_bf16_pairs(gamma[:, :64], gamma[:, 64:]),
         _pack_bf16_pairs(beta[:, :64], beta[:, 64:])), axis=1)
    beta_s = dot(x, wfsb_ref[...])
    gamma_s = dot(x, wfsg_ref[...])
    skip_ref[...] = jnp.maximum(gamma_s * dot(x, wskip_ref[...]) + beta_s, 0.0)


def _tc_pre1(x_aug, W_lin1a, W_film1a, W_skip1a, W_fskip1a):
    half = D_OUT
    return pl.pallas_call(
        _tc_pre1_body,
        grid=(NBLK, 2),
        in_specs=[
            pl.BlockSpec((B, KA), lambda n, c: (n, 0)),
            pl.BlockSpec((KA, half), lambda n, c: (0, c)),
            pl.BlockSpec((KA, half), lambda n, c: (0, c)),
            pl.BlockSpec((KA, half), lambda n, c: (0, 2 + c)),
            pl.BlockSpec((KA, half), lambda n, c: (0, c)),
            pl.BlockSpec((KA, half), lambda n, c: (0, c)),
            pl.BlockSpec((KA, half), lambda n, c: (0, 2 + c)),
        ],
        out_specs=[
            pl.BlockSpec((1, B, half), lambda n, c: (c, n, 0)),
            pl.BlockSpec((1, B, half), lambda n, c: (c, n, 0)),
            pl.BlockSpec((B, half), lambda n, c: (n, c)),
        ],
        out_shape=[
            jax.ShapeDtypeStruct((2, NPAD, half), jnp.float32),
            jax.ShapeDtypeStruct((2, NPAD, half), jnp.int32),
            jax.ShapeDtypeStruct((NPAD, H), jnp.float32),
        ],
    )(x_aug, W_lin1a, W_film1a, W_film1a, W_skip1a, W_fskip1a, W_fskip1a)


def _tc_mid_body(acc_ref, cnt_ref, skip_ref, wlin_ref, wfilm_ref, bf_ref,
                 wskip_ref, wfskip_ref,
                 xl2_ref, gb2_ref, skip2_ref):
    dot = functools.partial(jnp.dot, preferred_element_type=jnp.float32)
    summed = jnp.concatenate((acc_ref[0], acc_ref[1]), axis=1)
    cnt = jnp.sum(cnt_ref[:, 0], axis=(0, 1)).reshape(B, 1)
    h = jnp.maximum(skip_ref[...] + summed / jnp.clip(cnt, 1.0, None), 0.0)
    xl2_ref[...] = dot(h, wlin_ref[...])
    f2 = dot(h, wfilm_ref[...]) + bf_ref[...]
    gb2_ref[...] = jnp.concatenate(
        (_pack_bf16_pairs(f2[:, 128:192], f2[:, 192:]),
         _pack_bf16_pairs(f2[:, :64], f2[:, 64:128])), axis=1)
    fs2 = dot(h, wfskip_ref[...])
    skip2_ref[...] = jnp.maximum(
        fs2[:, 128:] * dot(h, wskip_ref[...]) + fs2[:, :128], 0.0)


def _tc_mid(acc1, cnt, skip1, W_lin2, W_film2, b_film2, W_skip2, W_fskip2):
    b2 = b_film2.reshape(1, 2 * D_OUT)
    return pl.pallas_call(
        _tc_mid_body,
        grid=(NBLK,),
        in_specs=[
            pl.BlockSpec((2, B, 128), lambda n: (0, n, 0)),
            pl.BlockSpec((2, 1, NSUB, B), lambda n: (0, n, 0, 0)),
            pl.BlockSpec((B, H), lambda n: (n, 0)),
            pl.BlockSpec((H, D_OUT), lambda n: (0, 0)),
            pl.BlockSpec((H, 2 * D_OUT), lambda n: (0, 0)),
            pl.BlockSpec((1, 2 * D_OUT), lambda n: (0, 0)),
            pl.BlockSpec((H, D_OUT), lambda n: (0, 0)),
            pl.BlockSpec((H, 2 * D_OUT), lambda n: (0, 0)),
        ],
        out_specs=[
            pl.BlockSpec((B, D_OUT), lambda n: (n, 0)),
            pl.BlockSpec((B, D_OUT), lambda n: (n, 0)),
            pl.BlockSpec((B, D_OUT), lambda n: (n, 0)),
        ],
        out_shape=[
            jax.ShapeDtypeStruct((NPAD, D_OUT), jnp.float32),
            jax.ShapeDtypeStruct((NPAD, D_OUT), jnp.int32),
            jax.ShapeDtypeStruct((NPAD, D_OUT), jnp.float32),
        ],
    )(acc1, cnt, skip1, W_lin2, W_film2, b2, W_skip2, W_fskip2)


def _tc_final_body(acc2_ref, skip2_ref, cnt_ref, out_ref):
    summed = acc2_ref[0] + acc2_ref[1]
    cnt = jnp.sum(cnt_ref[:, 0], axis=(0, 1)).reshape(B, 1)
    out_ref[...] = skip2_ref[...] + summed / jnp.clip(cnt, 1.0, None)


def _tc_final(acc2, skip2, cnt):
    return pl.pallas_call(
        _tc_final_body,
        grid=(NBLK,),
        in_specs=[
            pl.BlockSpec((2, B, D_OUT), lambda n: (0, n, 0)),
            pl.BlockSpec((B, D_OUT), lambda n: (n, 0)),
            pl.BlockSpec((2, 1, NSUB, B), lambda n: (0, n, 0, 0)),
        ],
        out_specs=pl.BlockSpec((B, D_OUT), lambda n: (n, 0)),
        out_shape=jax.ShapeDtypeStruct((NPAD, D_OUT), jnp.float32),
    )(acc2, skip2, cnt)


def _make_sc_agg(feature_split):
    mesh = plsc.VectorSubcoreMesh(
        core_axis_name="c", subcore_axis_name="s", num_cores=2,
        num_subcores=NSUB)
    chunks = CHUNKS1 if feature_split else CHUNKS2

    def body(epk_hbm, xl_hbm, gb_hbm, out_hbm,
             eidx0, eidx1, didxs0, didxs1, xl0, gb0, xl1, gb1,
             acc_sh, semx0, semg0, semx1, semg1, sems0, sems1):
        c = lax.axis_index("c")
        s = lax.axis_index("s")
        zeros16 = jnp.zeros((16,), jnp.float32)

        def zrow(i, carry):
            for j in range(128 // 16):
                xl0[i, pl.ds(j * 16, 16)] = zeros16
            return carry
        lax.fori_loop(0, C, zrow, 0)
        full = ROWS_PER_SUB // C
        for t in range(full):
            pltpu.sync_copy(xl0, acc_sh.at[pl.ds(s * ROWS_PER_SUB + t * C, C)])
        rem = ROWS_PER_SUB - full * C
        if rem:
            pltpu.sync_copy(
                xl0.at[pl.ds(0, rem)],
                acc_sh.at[pl.ds(s * ROWS_PER_SUB + full * C, rem)])
        plsc.subcore_barrier()

        if feature_split:
            base = s * (chunks * C)
            estride = C
            coff = c * NPAD
        else:
            base = s * (2 * chunks * C) + c * C
            estride = 2 * C
            coff = None

        def drain_scatter(xl_v, sems):
            pltpu.make_async_copy(xl_v, acc_sh.at[pl.ds(0, C)], sems).wait()

        def fetch(g, eidx_v, xl_v, gb_v, semx, semg,
                  sems=None, scat_pred=None):
            e0 = base + g * estride
            pltpu.sync_copy(epk_hbm.at[pl.ds(2 * e0, 2 * C)], eidx_v)
            if feature_split:
                def adj(i, cr):
                    eidx_v[pl.ds(i * 16, 16)] = eidx_v[pl.ds(i * 16, 16)] + coff
                    return cr
                lax.fori_loop(0, 2 * C // 16, adj, 0)
            if sems is not None:
                if scat_pred is None:
                    drain_scatter(xl_v, sems)
                else:
                    @pl.when(scat_pred)
                    def _():
                        drain_scatter(xl_v, sems)
            pltpu.async_copy(xl_hbm.at[eidx_v.at[pl.ds(0, C)]], xl_v, semx)
            pltpu.async_copy(gb_hbm.at[eidx_v.at[pl.ds(C, C)]], gb_v, semg)

        def drain(xl_v, gb_v, semx, semg):
            pltpu.make_async_copy(xl_hbm.at[pl.ds(0, C)], xl_v, semx).wait()
            pltpu.make_async_copy(gb_hbm.at[pl.ds(0, C)], gb_v, semg).wait()

        himask = jnp.int32(-65536)

        def compute(didx_v, xl_v, gb_v):
            def mrow(i, cr):
                for j in range(4):
                    wg = gb_v[i, pl.ds(j * 16, 16)]
                    wb = gb_v[i, pl.ds(64 + j * 16, 16)]
                    glo = plsc.bitcast(lax.shift_left(wg, 16), jnp.float32)
                    blo = plsc.bitcast(lax.shift_left(wb, 16), jnp.float32)
                    ghi = plsc.bitcast(lax.bitwise_and(wg, himask),
                                       jnp.float32)
                    bhi = plsc.bitcast(lax.bitwise_and(wb, himask),
                                       jnp.float32)
                    xlo = xl_v[i, pl.ds(32 * j, 16)]
                    xhi = xl_v[i, pl.ds(32 * j + 16, 16)]
                    xl_v[i, pl.ds(32 * j, 16)] = jnp.maximum(
                        glo * xlo + blo, 0.0)
                    xl_v[i, pl.ds(32 * j + 16, 16)] = jnp.maximum(
                        ghi * xhi + bhi, 0.0)
                return cr
            lax.fori_loop(0, C, mrow, 0)

        def scatter(eidx_v, didxs_v, xl_v, sems):
            def dcp(i, cr):
                d16 = eidx_v[pl.ds(C + i * 16, 16)]
                didxs_v[pl.ds(i * 16, 16)] = (d16 - coff) if feature_split else d16
                return cr
            lax.fori_loop(0, C // 16, dcp, 0)
            pltpu.make_async_copy(
                xl_v, acc_sh.at[didxs_v], sems).start(add=True)

        fetch(0, eidx0, xl0, gb0, semx0, semg0)

        def pair(g2, carry):
            g = 2 * g2
            fetch(g + 1, eidx1, xl1, gb1, semx1, semg1,
                  sems1, scat_pred=g2 > 0)
            drain(xl0, gb0, semx0, semg0)
            compute(eidx0, xl0, gb0)
            scatter(eidx0, didxs0, xl0, sems0)

            @pl.when(g + 2 < chunks)
            def _():
                fetch(g + 2, eidx0, xl0, gb0, semx0, semg0, sems0)
            drain(xl1, gb1, semx1, semg1)
            compute(eidx1, xl1, gb1)
            scatter(eidx1, didxs1, xl1, sems1)
            return carry
        lax.fori_loop(0, chunks // 2, pair, 0)
        if chunks % 2:
            drain(xl0, gb0, semx0, semg0)
            compute(eidx0, xl0, gb0)
            scatter(eidx0, didxs0, xl0, sems0)
        drain_scatter(xl0, sems0)
        drain_scatter(xl1, sems1)
        plsc.subcore_barrier()
        pltpu.sync_copy(acc_sh.at[pl.ds(s * ROWS_PER_SUB, ROWS_PER_SUB)],
                        out_hbm.at[c, pl.ds(s * ROWS_PER_SUB, ROWS_PER_SUB)])

    return pl.kernel(
        body,
        out_type=jax.ShapeDtypeStruct((2, NPAD, 128), jnp.float32),
        mesh=mesh,
        compiler_params=pltpu.CompilerParams(needs_layout_passes=False),
        scratch_types=(
            [pltpu.VMEM((2 * C,), jnp.int32)] * 2
            + [pltpu.VMEM((C,), jnp.int32)] * 2
            + [pltpu.VMEM((C, 128), jnp.float32),
               pltpu.VMEM((C, 128), jnp.int32)] * 2
            + [pltpu.VMEM_SHARED((H_ACC, 128), jnp.float32)]
            + [pltpu.SemaphoreType.DMA] * 6
        ),
    )


def _make_sc_count():
    mesh = plsc.VectorSubcoreMesh(
        core_axis_name="c", subcore_axis_name="s", num_cores=2,
        num_subcores=NSUB)

    def body(dst_hbm, cnt_hbm, didx_v, hist_v):
        c = lax.axis_index("c")
        s = lax.axis_index("s")
        w = c * NSUB + s
        zeros16 = jnp.zeros((16,), jnp.float32)
        iota16 = lax.iota(jnp.int32, 16)
        part16 = lax.bitwise_and(iota16, 3) * NPAD

        def hz(i, cr):
            hist_v[pl.ds(i * 16, 16)] = zeros16
            return cr
        lax.fori_loop(0, 4 * NPAD // 16, hz, 0)
        pltpu.sync_copy(dst_hbm.at[pl.ds(w * E_PER_TILE, E_PER_TILE)], didx_v)

        def step(i, cr):
            d16 = didx_v[pl.ds(i * 16, 16)]
            addr = part16 + d16
            for p in range(4):
                msk = jnp.logical_and(iota16 >= 4 * p, iota16 < 4 * p + 4)
                old = plsc.load_gather(hist_v, [addr], mask=msk)
                plsc.store_scatter(hist_v, [addr], old + 1.0, mask=msk)
            return cr
        lax.fori_loop(0, E_PER_TILE // 16, step, 0)

        def hred(i, cr):
            v = (hist_v[pl.ds(i * 16, 16)]
                 + hist_v[pl.ds(NPAD + i * 16, 16)]
                 + hist_v[pl.ds(2 * NPAD + i * 16, 16)]
                 + hist_v[pl.ds(3 * NPAD + i * 16, 16)])
            hist_v[pl.ds(i * 16, 16)] = v
            return cr
        lax.fori_loop(0, NPAD // 16, hred, 0)
        for n in range(NBLK):
            pltpu.sync_copy(hist_v.at[pl.ds(n * B, B)], cnt_hbm.at[c, n, s])

    return pl.kernel(
        body,
        out_type=jax.ShapeDtypeStruct((2, NBLK, NSUB, B), jnp.float32),
        mesh=mesh,
        compiler_params=pltpu.CompilerParams(needs_layout_passes=False),
        scratch_types=[
            pltpu.VMEM((E_PER_TILE,), jnp.int32),
            pltpu.VMEM((4 * NPAD,), jnp.float32),
        ],
    )


_sc_agg_cached = functools.lru_cache(maxsize=None)(_make_sc_agg)
_sc_count_cached = functools.lru_cache(maxsize=None)(_make_sc_count)


@jax.jit
def kernel(x, edge_index, W_lin1, W_film1, b_film1, W_skip1, W_fskip1,
           W_lin2, W_film2, b_film2, W_skip2, W_fskip2):
    x = x.astype(jnp.float32)
    src_pad = edge_index[0].astype(jnp.int32)
    dst_pad = edge_index[1].astype(jnp.int32)
    x_pad = jnp.pad(x, ((0, NPAD - N), (0, 0)))
    ones_col = jnp.ones((NPAD, 1), jnp.float32)
    x_aug = jnp.concatenate(
        [x_pad, ones_col, jnp.zeros((NPAD, KA - D_IN - 1), jnp.float32)], axis=1)

    def _aug_w(w, bias=None):
        pad = jnp.zeros((KA - D_IN, w.shape[1]), jnp.float32)
        if bias is not None:
            pad = pad.at[0].set(bias)
        return jnp.concatenate([w, pad], axis=0)

    def _perm(n):
        q = np.arange(n)
        b, r = q - q % 128, q % 128
        h, r2 = r // 64, r % 64
        return b + 32 * (r2 // 16) + 16 * h + r2 % 16

    p512 = _perm(512)
    xl1, gb1, skip1 = _tc_pre1(x_aug, _aug_w(W_lin1),
                               _aug_w(W_film1[:, p512], b_film1[p512]),
                               _aug_w(W_skip1), _aug_w(W_fskip1))
    cnt1 = _sc_count_cached()(dst_pad)
    epk = jnp.stack([src_pad.reshape(-1, C), dst_pad.reshape(-1, C)],
                    axis=1).reshape(-1)
    acc1 = _sc_agg_cached(True)(
        epk, xl1.reshape(2 * NPAD, 128), gb1.reshape(2 * NPAD, 128))
    pblk = np.concatenate([_perm(128), 128 + _perm(128)])
    xl2, gb2, skip2 = _tc_mid(acc1, cnt1, skip1, W_lin2,
                              W_film2[:, pblk], b_film2[pblk],
                              W_skip2, W_fskip2)
    acc2 = _sc_agg_cached(False)(epk, xl2, gb2)
    out = _tc_final(acc2, skip2, cnt1)
    return out[:N]
